# baseline (speedup 1.0000x reference)
import functools

import jax
import jax.numpy as jnp
import numpy as np
from jax import lax
from jax.experimental import pallas as pl
from jax.experimental.pallas import tpu as pltpu
from jax.experimental.pallas import tpu_sc as plsc

VOCAB = 1000000
EMBED = 16
B = 16384
L = 200
N_IDX = B * L
N_ELEM = N_IDX * EMBED


_ROTS = ((13, 15, 26, 6), (17, 29, 16, 24))


def _np_threefry2x32(k0, k1, x0, x1):
    m = 0xFFFFFFFF
    ks = (k0 & m, k1 & m, (k0 ^ k1 ^ 0x1BD11BDA) & m)
    x0 = (x0 + ks[0]) & m
    x1 = (x1 + ks[1]) & m
    for i in range(5):
        for r in _ROTS[i % 2]:
            x0 = (x0 + x1) & m
            x1 = ((x1 << r) | (x1 >> (32 - r))) & m
            x1 = x1 ^ x0
        x0 = (x0 + ks[(i + 1) % 3]) & m
        x1 = (x1 + ks[(i + 2) % 3] + i + 1) & m
    return x0, x1


def _np_fold_in(k0, k1, data):
    return _np_threefry2x32(k0, k1, (data >> 32) & 0xFFFFFFFF,
                            data & 0xFFFFFFFF)


_K0, _K1 = _np_fold_in(0, 0, 12345)


_NC = 2
_NS = 16
_NW = _NC * _NS

_XCOLS = 128
_XROWS = N_IDX // _XCOLS
_ROWS_PER_W = _XROWS // _NW
_CHUNK = 25
_NCHUNK = _ROWS_PER_W // _CHUNK


def _sc_gather(x1_hbm, table_hbm, out_hbm, idx_v, rows_v, sem):
    wid = lax.axis_index("s") * _NC + lax.axis_index("c")
    w_base = wid * _ROWS_PER_W

    def chunk_body(c, _):
        i0 = (w_base + c * _CHUNK) * _XCOLS
        pltpu.sync_copy(x1_hbm.at[pl.ds(i0, _CHUNK * _XCOLS)], idx_v)

        def fire(j, _):
            pltpu.async_copy(table_hbm.at[idx_v.at[pl.ds(j * _XCOLS, _XCOLS)]],
                             rows_v.at[pl.ds(j * _XCOLS, _XCOLS)], sem)
            return _

        lax.fori_loop(0, _CHUNK, fire, None)

        def drain(j, _):
            pltpu.make_async_copy(table_hbm.at[idx_v.at[pl.ds(0, _XCOLS)]],
                                  rows_v.at[pl.ds(0, _XCOLS)], sem).wait()
            return _

        lax.fori_loop(0, _CHUNK, drain, None)
        pltpu.sync_copy(rows_v, out_hbm.at[pl.ds(i0, _CHUNK * _XCOLS)])
        return _

    lax.fori_loop(0, _NCHUNK, chunk_body, None)


@functools.cache
def _sc_gather_call():
    return pl.kernel(
        _sc_gather,
        out_type=jax.ShapeDtypeStruct((N_IDX, EMBED), jnp.float32),
        mesh=plsc.VectorSubcoreMesh(core_axis_name="c", subcore_axis_name="s",
                                    num_cores=_NC, num_subcores=_NS),
        scratch_types=[
            pltpu.VMEM((_CHUNK * _XCOLS,), jnp.int32),
            pltpu.VMEM((_CHUNK * _XCOLS, EMBED), jnp.float32),
            pltpu.SemaphoreType.DMA,
        ],
        compiler_params=pltpu.CompilerParams(use_tc_tiling_on_sc=False,
                                             skip_device_barrier=True),
    )


_COLS = L * EMBED
_RB = 128
_LO = np.float32(np.nextafter(np.float32(-1.0), np.float32(0.0)))
_SCALE = np.float32(0.1 * np.sqrt(2.0))

_QPOLY = tuple(np.float32(c * 0.1) for c in (
    -0.0011786685790866613, 0.015818433836102486, -0.07585867494344711,
    0.14640192687511444, -0.10946287959814072, 0.3720836937427521,
    -0.00709370756521821, 1.253585934638977))


def _threefry_bits(cnt):
    k0 = jnp.uint32(_K0)
    k1 = jnp.uint32(_K1)
    ks2 = jnp.uint32(_K0 ^ _K1 ^ 0x1BD11BDA)
    ks = (k0, k1, ks2)
    x0 = jnp.full(cnt.shape, k0, jnp.uint32)
    x1 = cnt + k1
    for i in range(5):
        for r in _ROTS[i % 2]:
            x0 = x0 + x1
            x1 = (x1 << np.uint32(r)) | (x1 >> np.uint32(32 - r))
            x1 = x1 ^ x0
        x0 = x0 + ks[(i + 1) % 3]
        x1 = x1 + ks[(i + 2) % 3] + jnp.uint32(i + 1)
    return x0 ^ x1


def _horner(t, coeffs):
    acc = jnp.full(t.shape, np.float32(coeffs[0]), jnp.float32)
    for c in coeffs[1:]:
        acc = acc * t + np.float32(c)
    return acc


def _noise(cnt):
    bits = _threefry_bits(cnt)
    fb = (bits >> jnp.uint32(9)) | jnp.uint32(0x3F800000)
    f = lax.bitcast_convert_type(fb, jnp.float32) - np.float32(1.0)
    u = jnp.maximum(_LO, f * (np.float32(1.0) - _LO) + _LO)
    s = jnp.sqrt(-jnp.log1p(-u * u))
    return u * _horner(s, _QPOLY)


_BH = B // 2


def _noise_body_half(h):
    def body(out_ref):
        i = pl.program_id(0)
        base = (h * _BH + i * _RB) * _COLS
        row = lax.broadcasted_iota(jnp.int32, (_RB, _COLS), 0)
        col = lax.broadcasted_iota(jnp.int32, (_RB, _COLS), 1)
        cnt = (base + row * _COLS + col).astype(jnp.uint32)
        out_ref[...] = _noise(cnt)
    return body


def _noise_half_call(h):
    return pl.pallas_call(
        _noise_body_half(h),
        grid=(_BH // _RB,),
        out_specs=pl.BlockSpec((_RB, _COLS), lambda i: (i, 0)),
        out_shape=jax.ShapeDtypeStruct((_BH, _COLS), jnp.float32),
    )()


_NBLK = _BH // _RB


def _add_body(emb_ref, na_ref, nb_ref, out_ref):
    i = pl.program_id(0)
    noise = jnp.where(i < _NBLK, na_ref[...], nb_ref[...])
    out_ref[...] = jnp.clip(emb_ref[...] + noise,
                            np.float32(-1.0), np.float32(1.0))


_add_call = pl.pallas_call(
    _add_body,
    grid=(B // _RB,),
    in_specs=[pl.BlockSpec((_RB, _COLS), lambda i: (i, 0)),
              pl.BlockSpec((_RB, _COLS),
                           lambda i: (jnp.minimum(i, _NBLK - 1), 0)),
              pl.BlockSpec((_RB, _COLS),
                           lambda i: (jnp.maximum(i - _NBLK, 0), 0))],
    out_specs=pl.BlockSpec((_RB, _COLS), lambda i: (i, 0)),
    out_shape=jax.ShapeDtypeStruct((B, _COLS), jnp.float32),
)


def kernel(x, table):
    x1 = jnp.asarray(x, jnp.int32).reshape(-1)
    noise_a = _noise_half_call(0)
    table_g, noise_a = lax.optimization_barrier((table, noise_a))
    gathered = _sc_gather_call()(x1, table_g)
    noise_b = _noise_half_call(1)
    emb = gathered.reshape(B, _COLS)
    out = _add_call(emb, noise_a, noise_b)
    return out.reshape(B, L, EMBED)


# speedup vs baseline: 1.0604x; 1.0604x over previous
import functools

import jax
import jax.numpy as jnp
import numpy as np
from jax import lax
from jax.experimental import pallas as pl
from jax.experimental.pallas import tpu as pltpu
from jax.experimental.pallas import tpu_sc as plsc

VOCAB = 1000000
EMBED = 16
B = 16384
L = 200
N_IDX = B * L
N_ELEM = N_IDX * EMBED


_ROTS = ((13, 15, 26, 6), (17, 29, 16, 24))


def _np_threefry2x32(k0, k1, x0, x1):
    m = 0xFFFFFFFF
    ks = (k0 & m, k1 & m, (k0 ^ k1 ^ 0x1BD11BDA) & m)
    x0 = (x0 + ks[0]) & m
    x1 = (x1 + ks[1]) & m
    for i in range(5):
        for r in _ROTS[i % 2]:
            x0 = (x0 + x1) & m
            x1 = ((x1 << r) | (x1 >> (32 - r))) & m
            x1 = x1 ^ x0
        x0 = (x0 + ks[(i + 1) % 3]) & m
        x1 = (x1 + ks[(i + 2) % 3] + i + 1) & m
    return x0, x1


def _np_fold_in(k0, k1, data):
    return _np_threefry2x32(k0, k1, (data >> 32) & 0xFFFFFFFF,
                            data & 0xFFFFFFFF)


_K0, _K1 = _np_fold_in(0, 0, 12345)


_NC = 2
_NS = 16
_NW = _NC * _NS

_XCOLS = 128
_XROWS = N_IDX // _XCOLS
_ROWS_PER_W = _XROWS // _NW
_CHUNK = 25
_NCHUNK = _ROWS_PER_W // _CHUNK


def _sc_gather(x1_hbm, table_hbm, out_hbm, idx_v, rows_v, sem):
    wid = lax.axis_index("s") * _NC + lax.axis_index("c")
    w_base = wid * _ROWS_PER_W

    def chunk_body(c, _):
        i0 = (w_base + c * _CHUNK) * _XCOLS
        pltpu.sync_copy(x1_hbm.at[pl.ds(i0, _CHUNK * _XCOLS)], idx_v)

        def fire(j, _):
            pltpu.async_copy(table_hbm.at[idx_v.at[pl.ds(j * _XCOLS, _XCOLS)]],
                             rows_v.at[pl.ds(j * _XCOLS, _XCOLS)], sem)
            return _

        lax.fori_loop(0, _CHUNK, fire, None)

        def drain(j, _):
            pltpu.make_async_copy(table_hbm.at[idx_v.at[pl.ds(0, _XCOLS)]],
                                  rows_v.at[pl.ds(0, _XCOLS)], sem).wait()
            return _

        lax.fori_loop(0, _CHUNK, drain, None)
        pltpu.sync_copy(rows_v, out_hbm.at[pl.ds(i0, _CHUNK * _XCOLS)])
        return _

    lax.fori_loop(0, _NCHUNK, chunk_body, None)


@functools.cache
def _sc_gather_call():
    return pl.kernel(
        _sc_gather,
        out_type=jax.ShapeDtypeStruct((N_IDX, EMBED), jnp.float32),
        mesh=plsc.VectorSubcoreMesh(core_axis_name="c", subcore_axis_name="s",
                                    num_cores=_NC, num_subcores=_NS),
        scratch_types=[
            pltpu.VMEM((_CHUNK * _XCOLS,), jnp.int32),
            pltpu.VMEM((_CHUNK * _XCOLS, EMBED), jnp.float32),
            pltpu.SemaphoreType.DMA,
        ],
        compiler_params=pltpu.CompilerParams(use_tc_tiling_on_sc=False,
                                             skip_device_barrier=True),
    )


_COLS = L * EMBED
_RB = 128
_LO = np.float32(np.nextafter(np.float32(-1.0), np.float32(0.0)))
_SCALE = np.float32(0.1 * np.sqrt(2.0))

_QPOLY = tuple(np.float32(c * 0.1) for c in (
    -0.0011786685790866613, 0.015818433836102486, -0.07585867494344711,
    0.14640192687511444, -0.10946287959814072, 0.3720836937427521,
    -0.00709370756521821, 1.253585934638977))


def _threefry_bits(cnt):
    k0 = jnp.uint32(_K0)
    k1 = jnp.uint32(_K1)
    ks2 = jnp.uint32(_K0 ^ _K1 ^ 0x1BD11BDA)
    ks = (k0, k1, ks2)
    x0 = jnp.full(cnt.shape, k0, jnp.uint32)
    x1 = cnt + k1
    for i in range(5):
        for r in _ROTS[i % 2]:
            x0 = x0 + x1
            x1 = (x1 << np.uint32(r)) | (x1 >> np.uint32(32 - r))
            x1 = x1 ^ x0
        x0 = x0 + ks[(i + 1) % 3]
        x1 = x1 + ks[(i + 2) % 3] + jnp.uint32(i + 1)
    return x0 ^ x1


def _horner(t, coeffs):
    acc = jnp.full(t.shape, np.float32(coeffs[0]), jnp.float32)
    for c in coeffs[1:]:
        acc = acc * t + np.float32(c)
    return acc


def _noise(cnt):
    bits = _threefry_bits(cnt)
    fb = (bits >> jnp.uint32(9)) | jnp.uint32(0x3F800000)
    f = lax.bitcast_convert_type(fb, jnp.float32) - np.float32(1.0)
    u = jnp.maximum(_LO, f * (np.float32(1.0) - _LO) + _LO)
    s = jnp.sqrt(-jnp.log1p(-u * u))
    return u * _horner(s, _QPOLY)


def _noise_body(out_ref):
    i = pl.program_id(0)
    base = i * (_RB * _COLS)
    row = lax.broadcasted_iota(jnp.int32, (_RB, _COLS), 0)
    col = lax.broadcasted_iota(jnp.int32, (_RB, _COLS), 1)
    cnt = (base + row * _COLS + col).astype(jnp.uint32)
    out_ref[...] = _noise(cnt)


_noise_call = pl.pallas_call(
    _noise_body,
    grid=(B // _RB,),
    out_specs=pl.BlockSpec((_RB, _COLS), lambda i: (i, 0)),
    out_shape=jax.ShapeDtypeStruct((B, _COLS), jnp.float32),
)


def _add_body(emb_ref, noise_ref, out_ref):
    out_ref[...] = jnp.clip(emb_ref[...] + noise_ref[...],
                            np.float32(-1.0), np.float32(1.0))


_add_call = pl.pallas_call(
    _add_body,
    grid=(B // _RB,),
    in_specs=[pl.BlockSpec((_RB, _COLS), lambda i: (i, 0)),
              pl.BlockSpec((_RB, _COLS), lambda i: (i, 0))],
    out_specs=pl.BlockSpec((_RB, _COLS), lambda i: (i, 0)),
    out_shape=jax.ShapeDtypeStruct((B, _COLS), jnp.float32),
)


def kernel(x, table):
    x1 = jnp.asarray(x, jnp.int32).reshape(-1)
    gathered = _sc_gather_call()(x1, table)
    noise = _noise_call()
    emb = gathered.reshape(B, _COLS)
    out = _add_call(emb, noise)
    return out.reshape(B, L, EMBED)
